# Initial kernel scaffold; baseline (speedup 1.0000x reference)
import jax
import jax.numpy as jnp
from jax.experimental import pallas as pl


def kernel(x, props):
    raise NotImplementedError("write your pallas kernel here")


# speedup vs baseline: 1.1405x; 1.1405x over previous
import jax
import jax.numpy as jnp
from jax.experimental import pallas as pl

N = 64
NUM_TABLES = 7
H_BLK = 256
NEG = -1e30


def _map_props_kernel(x_ref, g1_ref, g2_ref, map_ref, props_ref):
    xb = x_ref[0]
    hb = xb.shape[0]

    s_iota = jax.lax.broadcasted_iota(jnp.int32, (hb, N, N), 1)
    e_iota = jax.lax.broadcasted_iota(jnp.int32, (hb, N, N), 2)
    tri = e_iota >= s_iota
    m = jnp.where(tri, xb[:, None, :], NEG)
    sh = 1
    while sh < N:
        shifted = jnp.concatenate(
            [jnp.full((hb, N, sh), NEG, jnp.float32), m[:, :, :-sh]], axis=2)
        m = jnp.maximum(m, shifted)
        sh *= 2
    map_ref[0] = jnp.where(tri, m, 0.0)

    tables = [xb]
    w = xb
    for k in range(NUM_TABLES - 1):
        step = 1 << k
        shifted = jnp.concatenate(
            [w[:, step:], jnp.full((hb, step), NEG, jnp.float32)], axis=1)
        w = jnp.maximum(w, shifted)
        tables.append(w)
    wt = jnp.concatenate(tables, axis=1)

    g1 = jax.lax.dot_general(g1_ref[...], wt, (((1,), (1,)), ((), ())),
                             preferred_element_type=jnp.float32)
    g2 = jax.lax.dot_general(g2_ref[...], wt, (((1,), (1,)), ((), ())),
                             preferred_element_type=jnp.float32)
    props_ref[0] = jnp.maximum(g1, g2)


def kernel(x, props):
    B, H, n = x.shape
    assert n == N
    P = props.shape[0]

    idx0 = props[:, 0].astype(jnp.int32)
    idx1 = (props[:, 1].astype(jnp.int32) - 1) % N
    valid = idx1 >= idx0
    length = idx1 - idx0 + 1
    k = ((length >= 2).astype(jnp.int32) + (length >= 4) + (length >= 8)
         + (length >= 16) + (length >= 32) + (length >= 64))
    row1 = k * N + idx0
    row2 = k * N + (idx1 - (1 << k) + 1)
    rows = jnp.arange(NUM_TABLES * N, dtype=jnp.int32)
    g1 = ((rows[None, :] == row1[:, None]) & valid[:, None]).astype(jnp.float32)
    g2 = ((rows[None, :] == row2[:, None]) & valid[:, None]).astype(jnp.float32)

    map_h, props_h = pl.pallas_call(
        _map_props_kernel,
        grid=(B, H // H_BLK),
        in_specs=[
            pl.BlockSpec((1, H_BLK, N), lambda b, h: (b, h, 0)),
            pl.BlockSpec((P, NUM_TABLES * N), lambda b, h: (0, 0)),
            pl.BlockSpec((P, NUM_TABLES * N), lambda b, h: (0, 0)),
        ],
        out_specs=[
            pl.BlockSpec((1, H_BLK, N, N), lambda b, h: (b, h, 0, 0)),
            pl.BlockSpec((1, P, H_BLK), lambda b, h: (b, 0, h)),
        ],
        out_shape=[
            jax.ShapeDtypeStruct((B, H, N, N), jnp.float32),
            jax.ShapeDtypeStruct((B, P, H), jnp.float32),
        ],
    )(x, g1, g2)

    tri = (jnp.arange(N)[:, None] <= jnp.arange(N)[None, :]).astype(x.dtype)
    map_mask = jnp.broadcast_to(tri[None, None], (B, 1, N, N))
    return props_h, map_h, map_mask


# speedup vs baseline: 1.9062x; 1.6713x over previous
import jax
import jax.numpy as jnp
from jax.experimental import pallas as pl
from jax.experimental.pallas import tpu as pltpu

N = 64
NUM_TABLES = 7
H_BLK = 256
NEG = -1e30


def _map_props_kernel(x_ref, g1_ref, g2_ref, map_ref, props_ref):
    xb = x_ref[0]
    hb = xb.shape[0]

    c_io = jax.lax.broadcasted_iota(jnp.int32, (hb, N * N), 1)
    e_io = jnp.bitwise_and(c_io, N - 1)
    s_io = jnp.right_shift(c_io, 6)
    tri = e_io >= s_io
    xt = pltpu.repeat(xb, N, axis=1)
    m = jnp.where(tri, xt, NEG)
    sh = 1
    while sh < N:
        rolled = pltpu.roll(m, sh, axis=1)
        m = jnp.maximum(m, jnp.where(e_io >= sh, rolled, NEG))
        sh *= 2
    map_ref[0] = jnp.where(tri, m, 0.0)

    tables = [xb]
    w = xb
    for k in range(NUM_TABLES - 1):
        step = 1 << k
        shifted = jnp.concatenate(
            [w[:, step:], jnp.full((hb, step), NEG, jnp.float32)], axis=1)
        w = jnp.maximum(w, shifted)
        tables.append(w)
    wt = jnp.concatenate(tables, axis=1)

    g1 = jax.lax.dot_general(g1_ref[...], wt, (((1,), (1,)), ((), ())),
                             preferred_element_type=jnp.float32)
    g2 = jax.lax.dot_general(g2_ref[...], wt, (((1,), (1,)), ((), ())),
                             preferred_element_type=jnp.float32)
    props_ref[0] = jnp.maximum(g1, g2)


def kernel(x, props):
    B, H, n = x.shape
    assert n == N
    P = props.shape[0]

    idx0 = props[:, 0].astype(jnp.int32)
    idx1 = (props[:, 1].astype(jnp.int32) - 1) % N
    valid = idx1 >= idx0
    length = idx1 - idx0 + 1
    k = ((length >= 2).astype(jnp.int32) + (length >= 4) + (length >= 8)
         + (length >= 16) + (length >= 32) + (length >= 64))
    row1 = k * N + idx0
    row2 = k * N + (idx1 - (1 << k) + 1)
    rows = jnp.arange(NUM_TABLES * N, dtype=jnp.int32)
    g1 = ((rows[None, :] == row1[:, None]) & valid[:, None]).astype(jnp.float32)
    g2 = ((rows[None, :] == row2[:, None]) & valid[:, None]).astype(jnp.float32)

    map_h, props_h = pl.pallas_call(
        _map_props_kernel,
        grid=(B, H // H_BLK),
        in_specs=[
            pl.BlockSpec((1, H_BLK, N), lambda b, h: (b, h, 0)),
            pl.BlockSpec((P, NUM_TABLES * N), lambda b, h: (0, 0)),
            pl.BlockSpec((P, NUM_TABLES * N), lambda b, h: (0, 0)),
        ],
        out_specs=[
            pl.BlockSpec((1, H_BLK, N * N), lambda b, h: (b, h, 0)),
            pl.BlockSpec((1, P, H_BLK), lambda b, h: (b, 0, h)),
        ],
        out_shape=[
            jax.ShapeDtypeStruct((B, H, N * N), jnp.float32),
            jax.ShapeDtypeStruct((B, P, H), jnp.float32),
        ],
    )(x, g1, g2)
    map_h = map_h.reshape(B, H, N, N)

    tri = (jnp.arange(N)[:, None] <= jnp.arange(N)[None, :]).astype(x.dtype)
    map_mask = jnp.broadcast_to(tri[None, None], (B, 1, N, N))
    return props_h, map_h, map_mask
